# speedup vs baseline: 36.2071x; 36.2071x over previous
import functools

import jax
import jax.numpy as jnp
from jax import lax
from jax.experimental import pallas as pl
from jax.experimental.pallas import tpu as pltpu
from jax.experimental.pallas import tpu_sc as plsc

_NC = 2
_NS = 16
_L = 16


def _sc_scatter_partials(idx2, upd2, Mp):
    nrows = idx2.shape[0]
    NW = _NC * _NS
    CH = 128
    rows_per_w = nrows // NW
    nblk = rows_per_w // CH
    ZB = 16384
    per_s = Mp // _NS

    mesh = plsc.VectorSubcoreMesh(
        core_axis_name="c", subcore_axis_name="s",
        num_cores=_NC, num_subcores=_NS)

    @functools.partial(
        pl.kernel,
        out_type=[jax.ShapeDtypeStruct((Mp,), jnp.float32),
                  jax.ShapeDtypeStruct((Mp,), jnp.float32)],
        mesh=mesh,
        scratch_types=[
            pltpu.VMEM((CH, 128), jnp.int32),
            pltpu.VMEM((CH, 128), jnp.float32),
            pltpu.VMEM((ZB,), jnp.float32),
            pltpu.VMEM_SHARED((Mp,), jnp.float32),
        ],
    )
    def k(idx_hbm, upd_hbm, out0, out1, idx_v, upd_v, zero_v, acc):
        c = lax.axis_index("c")
        s = lax.axis_index("s")
        w = c * _NS + s

        def zstore(i, carry):
            zero_v[pl.ds(i * _L, _L)] = jnp.zeros((_L,), jnp.float32)
            return carry
        lax.fori_loop(0, ZB // _L, zstore, 0)

        def zdma(i, carry):
            pltpu.sync_copy(zero_v, acc.at[pl.ds(s * per_s + i * ZB, ZB)])
            return carry
        lax.fori_loop(0, per_s // ZB, zdma, 0)
        plsc.subcore_barrier()

        def blk(b, carry):
            row0 = (w * nblk + b) * CH
            pltpu.sync_copy(idx_hbm.at[pl.ds(row0, CH)], idx_v)
            pltpu.sync_copy(upd_hbm.at[pl.ds(row0, CH)], upd_v)

            def tr(j, carry2):
                pltpu.sync_copy(upd_v.at[j], acc.at[idx_v.at[j]], add=True)
                return carry2
            lax.fori_loop(0, CH, tr, 0)
            return carry
        lax.fori_loop(0, nblk, blk, 0)
        plsc.subcore_barrier()

        @pl.when(c == 0)
        def _():
            pltpu.sync_copy(acc.at[pl.ds(s * per_s, per_s)],
                            out0.at[pl.ds(s * per_s, per_s)])

        @pl.when(c == 1)
        def _():
            pltpu.sync_copy(acc.at[pl.ds(s * per_s, per_s)],
                            out1.at[pl.ds(s * per_s, per_s)])

    return k(idx2, upd2)


def _combine(d2, a2, b2):
    R, C = d2.shape
    BR = R // 8

    def body(d_ref, a_ref, b_ref, o_ref):
        o_ref[...] = d_ref[...] + a_ref[...] + b_ref[...]

    return pl.pallas_call(
        body,
        grid=(R // BR,),
        in_specs=[pl.BlockSpec((BR, C), lambda i: (i, 0))] * 3,
        out_specs=pl.BlockSpec((BR, C), lambda i: (i, 0)),
        out_shape=jax.ShapeDtypeStruct((R, C), jnp.float32),
    )(d2, a2, b2)


def kernel(data, indices, updates):
    M = data.shape[0]
    N = indices.shape[0]
    Mp = 1 << 20
    idx2 = indices.astype(jnp.int32).reshape(N // 128, 128)
    upd2 = updates.reshape(N // 128, 128)
    q0, q1 = _sc_scatter_partials(idx2, upd2, Mp)
    dp = jnp.pad(data, (0, Mp - M))
    out2 = _combine(dp.reshape(Mp // 128, 128),
                    q0.reshape(Mp // 128, 128),
                    q1.reshape(Mp // 128, 128))
    return out2.reshape(Mp)[:M]


# speedup vs baseline: 59.0575x; 1.6311x over previous
import functools

import jax
import jax.numpy as jnp
from jax import lax
from jax.experimental import pallas as pl
from jax.experimental.pallas import tpu as pltpu
from jax.experimental.pallas import tpu_sc as plsc

_NC = 2
_NS = 16
_L = 16


def _sc_scatter_partials(idx1, upd1, Mp):
    n = idx1.shape[0]
    NW = _NC * _NS
    B = 16384
    per_w = n // NW
    nblk = per_w // B
    ZB = 16384
    per_s = Mp // _NS

    mesh = plsc.VectorSubcoreMesh(
        core_axis_name="c", subcore_axis_name="s",
        num_cores=_NC, num_subcores=_NS)

    @functools.partial(
        pl.kernel,
        out_type=[jax.ShapeDtypeStruct((Mp,), jnp.float32),
                  jax.ShapeDtypeStruct((Mp,), jnp.float32)],
        mesh=mesh,
        scratch_types=[
            pltpu.VMEM((B,), jnp.int32),
            pltpu.VMEM((B,), jnp.float32),
            pltpu.VMEM((ZB,), jnp.float32),
            pltpu.VMEM_SHARED((Mp,), jnp.float32),
        ],
    )
    def k(idx_hbm, upd_hbm, out0, out1, idx_v, upd_v, zero_v, acc):
        c = lax.axis_index("c")
        s = lax.axis_index("s")
        w = c * _NS + s

        def zstore(i, carry):
            zero_v[pl.ds(i * _L, _L)] = jnp.zeros((_L,), jnp.float32)
            return carry
        lax.fori_loop(0, ZB // _L, zstore, 0)

        def zdma(i, carry):
            pltpu.sync_copy(zero_v, acc.at[pl.ds(s * per_s + i * ZB, ZB)])
            return carry
        lax.fori_loop(0, per_s // ZB, zdma, 0)
        plsc.subcore_barrier()

        def blk(b, carry):
            base = (w * nblk + b) * B
            pltpu.sync_copy(idx_hbm.at[pl.ds(base, B)], idx_v)
            pltpu.sync_copy(upd_hbm.at[pl.ds(base, B)], upd_v)
            pltpu.sync_copy(upd_v, acc.at[idx_v], add=True)
            return carry
        lax.fori_loop(0, nblk, blk, 0)
        plsc.subcore_barrier()

        @pl.when(c == 0)
        def _():
            pltpu.sync_copy(acc.at[pl.ds(s * per_s, per_s)],
                            out0.at[pl.ds(s * per_s, per_s)])

        @pl.when(c == 1)
        def _():
            pltpu.sync_copy(acc.at[pl.ds(s * per_s, per_s)],
                            out1.at[pl.ds(s * per_s, per_s)])

    return k(idx1, upd1)


def _combine(d2, a2, b2):
    R, C = d2.shape
    BR = R // 8

    def body(d_ref, a_ref, b_ref, o_ref):
        o_ref[...] = d_ref[...] + a_ref[...] + b_ref[...]

    return pl.pallas_call(
        body,
        grid=(R // BR,),
        in_specs=[pl.BlockSpec((BR, C), lambda i: (i, 0))] * 3,
        out_specs=pl.BlockSpec((BR, C), lambda i: (i, 0)),
        out_shape=jax.ShapeDtypeStruct((R, C), jnp.float32),
    )(d2, a2, b2)


def kernel(data, indices, updates):
    M = data.shape[0]
    N = indices.shape[0]
    Mp = 1 << 20
    del N
    q0, q1 = _sc_scatter_partials(indices.astype(jnp.int32), updates, Mp)
    dp = jnp.pad(data, (0, Mp - M))
    out2 = _combine(dp.reshape(Mp // 128, 128),
                    q0.reshape(Mp // 128, 128),
                    q1.reshape(Mp // 128, 128))
    return out2.reshape(Mp)[:M]


# speedup vs baseline: 72.3169x; 1.2245x over previous
import functools

import jax
import jax.numpy as jnp
from jax import lax
from jax.experimental import pallas as pl
from jax.experimental.pallas import tpu as pltpu
from jax.experimental.pallas import tpu_sc as plsc

_NC = 2
_NS = 16
_L = 16


def _sc_scatter_partials(idx1, upd1, Mp):
    n = idx1.shape[0]
    NW = _NC * _NS
    B = 8192
    per_w = n // NW
    nblk = per_w // B
    ZB = 16384
    per_s = Mp // _NS

    mesh = plsc.VectorSubcoreMesh(
        core_axis_name="c", subcore_axis_name="s",
        num_cores=_NC, num_subcores=_NS)

    @functools.partial(
        pl.kernel,
        out_type=[jax.ShapeDtypeStruct((Mp,), jnp.float32),
                  jax.ShapeDtypeStruct((Mp,), jnp.float32)],
        mesh=mesh,
        scratch_types=[
            pltpu.VMEM((B,), jnp.int32),
            pltpu.VMEM((B,), jnp.float32),
            pltpu.VMEM((B,), jnp.int32),
            pltpu.VMEM((B,), jnp.float32),
            pltpu.VMEM((ZB,), jnp.float32),
            pltpu.VMEM_SHARED((Mp,), jnp.float32),
            pltpu.SemaphoreType.DMA,
            pltpu.SemaphoreType.DMA,
        ],
    )
    def k(idx_hbm, upd_hbm, out0, out1, idx_a, upd_a, idx_b, upd_b,
          zero_v, acc, sem_a, sem_b):
        c = lax.axis_index("c")
        s = lax.axis_index("s")
        w = c * _NS + s

        bufs = ((idx_a, upd_a, sem_a), (idx_b, upd_b, sem_b))

        def start(b, iv, uv, sem):
            base = (w * nblk + b) * B
            ci = pltpu.async_copy(idx_hbm.at[pl.ds(base, B)], iv, sem)
            cu = pltpu.async_copy(upd_hbm.at[pl.ds(base, B)], uv, sem)
            return ci, cu

        pend = start(0, *bufs[0])

        def zstore(i, carry):
            zero_v[pl.ds(i * _L, _L)] = jnp.zeros((_L,), jnp.float32)
            return carry
        lax.fori_loop(0, ZB // _L, zstore, 0)

        def zdma(i, carry):
            pltpu.sync_copy(zero_v, acc.at[pl.ds(s * per_s + i * ZB, ZB)])
            return carry
        lax.fori_loop(0, per_s // ZB, zdma, 0)
        plsc.subcore_barrier()

        for b in range(nblk):
            iv, uv, _ = bufs[b % 2]
            pend[0].wait()
            pend[1].wait()
            if b + 1 < nblk:
                pend = start(b + 1, *bufs[(b + 1) % 2])
            pltpu.sync_copy(uv, acc.at[iv], add=True)
        plsc.subcore_barrier()

        @pl.when(c == 0)
        def _():
            pltpu.sync_copy(acc.at[pl.ds(s * per_s, per_s)],
                            out0.at[pl.ds(s * per_s, per_s)])

        @pl.when(c == 1)
        def _():
            pltpu.sync_copy(acc.at[pl.ds(s * per_s, per_s)],
                            out1.at[pl.ds(s * per_s, per_s)])

    return k(idx1, upd1)


def _combine(d2, a2, b2):
    R, C = d2.shape
    BR = R // 8

    def body(d_ref, a_ref, b_ref, o_ref):
        o_ref[...] = d_ref[...] + a_ref[...] + b_ref[...]

    return pl.pallas_call(
        body,
        grid=(R // BR,),
        in_specs=[pl.BlockSpec((BR, C), lambda i: (i, 0))] * 3,
        out_specs=pl.BlockSpec((BR, C), lambda i: (i, 0)),
        out_shape=jax.ShapeDtypeStruct((R, C), jnp.float32),
    )(d2, a2, b2)


def kernel(data, indices, updates):
    M = data.shape[0]
    N = indices.shape[0]
    Mp = 1 << 20
    del N
    q0, q1 = _sc_scatter_partials(indices.astype(jnp.int32), updates, Mp)
    dp = jnp.pad(data, (0, Mp - M))
    out2 = _combine(dp.reshape(Mp // 128, 128),
                    q0.reshape(Mp // 128, 128),
                    q1.reshape(Mp // 128, 128))
    return out2.reshape(Mp)[:M]


# speedup vs baseline: 77.6172x; 1.0733x over previous
import functools

import jax
import jax.numpy as jnp
from jax import lax
from jax.experimental import pallas as pl
from jax.experimental.pallas import tpu as pltpu
from jax.experimental.pallas import tpu_sc as plsc

_NC = 2
_NS = 16
_L = 16


def _sc_scatter_partials(idx1, upd1, Mp):
    n = idx1.shape[0]
    NW = _NC * _NS
    B = 8192
    per_w = n // NW
    nblk = per_w // B
    ZB = 16384
    per_s = Mp // _NS

    mesh = plsc.VectorSubcoreMesh(
        core_axis_name="c", subcore_axis_name="s",
        num_cores=_NC, num_subcores=_NS)

    @functools.partial(
        pl.kernel,
        out_type=[jax.ShapeDtypeStruct((Mp,), jnp.float32),
                  jax.ShapeDtypeStruct((Mp,), jnp.float32)],
        mesh=mesh,
        scratch_types=[
            pltpu.VMEM((B,), jnp.int32),
            pltpu.VMEM((B,), jnp.float32),
            pltpu.VMEM((B,), jnp.int32),
            pltpu.VMEM((B,), jnp.float32),
            pltpu.VMEM((ZB,), jnp.float32),
            pltpu.VMEM_SHARED((Mp,), jnp.float32),
            pltpu.SemaphoreType.DMA,
            pltpu.SemaphoreType.DMA,
        ],
    )
    def k(idx_hbm, upd_hbm, out0, out1, idx_a, upd_a, idx_b,
          upd_b, zero_v, acc, sem_a, sem_b):
        c = lax.axis_index("c")
        s = lax.axis_index("s")
        w = c * _NS + s

        bufs = ((idx_a, upd_a, sem_a), (idx_b, upd_b, sem_b))

        def start(b, iv, uv, sem):
            base = (w * nblk + b) * B
            ci = pltpu.async_copy(idx_hbm.at[pl.ds(base, B)], iv, sem)
            cu = pltpu.async_copy(upd_hbm.at[pl.ds(base, B)], uv, sem)
            return ci, cu

        pend = start(0, *bufs[0])

        def zstore(i, carry):
            zero_v[pl.ds(i * _L, _L)] = jnp.zeros((_L,), jnp.float32)
            return carry
        lax.fori_loop(0, ZB // _L, zstore, 0)

        def zdma(i, carry):
            pltpu.sync_copy(zero_v, acc.at[pl.ds(s * per_s + i * ZB, ZB)])
            return carry
        lax.fori_loop(0, per_s // ZB, zdma, 0)

        plsc.subcore_barrier()

        for b in range(nblk):
            iv, uv, _ = bufs[b % 2]
            pend[0].wait()
            pend[1].wait()
            if b + 1 < nblk:
                pend = start(b + 1, *bufs[(b + 1) % 2])
            pltpu.sync_copy(uv, acc.at[iv], add=True)
        plsc.subcore_barrier()

        @pl.when(c == 0)
        def _():
            pltpu.sync_copy(acc.at[pl.ds(s * per_s, per_s)],
                            out0.at[pl.ds(s * per_s, per_s)])

        @pl.when(c == 1)
        def _():
            pltpu.sync_copy(acc.at[pl.ds(s * per_s, per_s)],
                            out1.at[pl.ds(s * per_s, per_s)])

    return k(idx1, upd1)


def _combine(d, a, b):
    M = d.shape[0]
    Mp = a.shape[0]
    BLK = Mp // 8

    def body(d_ref, a_ref, b_ref, o_ref):
        o_ref[...] = d_ref[...] + a_ref[...] + b_ref[...]

    return pl.pallas_call(
        body,
        grid=(Mp // BLK,),
        in_specs=[pl.BlockSpec((BLK,), lambda i: (i,))] * 3,
        out_specs=pl.BlockSpec((BLK,), lambda i: (i,)),
        out_shape=jax.ShapeDtypeStruct((M,), jnp.float32),
    )(d, a, b)


def kernel(data, indices, updates):
    Mp = 1 << 20
    q0, q1 = _sc_scatter_partials(indices.astype(jnp.int32), updates, Mp)
    return _combine(data, q0, q1)
